# Initial kernel scaffold; baseline (speedup 1.0000x reference)
import functools

import jax
import jax.numpy as jnp
from jax import lax
from jax.experimental import pallas as pl
from jax.experimental.pallas import tpu as pltpu
from jax.experimental.pallas import tpu_sc as plsc

N_FEAT = 1024
DIM = 128
N_CLASSES = 100000
CHUNK = 1000
N_CHUNKS = N_CLASSES // CHUNK
TOPK = 8
INV_TEMP = 1.0 / 0.07

NUM_WORKERS = 32
ROWS_PER_W = N_FEAT // NUM_WORKERS


def _extract_top8(vals, idxs):
    outv, outi = [], []
    big = jnp.int32(0x7FFFFFFF)
    for _ in range(TOPK):
        m = jnp.max(vals, axis=1, keepdims=True)
        im = jnp.min(jnp.where(vals == m, idxs, big), axis=1, keepdims=True)
        outv.append(m)
        outi.append(im)
        vals = jnp.where(idxs == im, jnp.float32(-2.0), vals)
    return jnp.concatenate(outv, axis=1), jnp.concatenate(outi, axis=1)


def _topk_tc_body(feat_ref, p_ref, topv_ref, topi_ref,
                  qn_ref, runv_ref, runi_ref, zsum_ref):
    c = pl.program_id(0)

    @pl.when(c == 0)
    def _init():
        q = feat_ref[...]
        n = jnp.sqrt(jnp.sum(q * q, axis=1, keepdims=True))
        qn_ref[...] = q / jnp.maximum(n, jnp.float32(1e-12))
        runv_ref[...] = jnp.full((N_FEAT, TOPK), -1.0, jnp.float32)
        runi_ref[...] = jnp.zeros((N_FEAT, TOPK), jnp.int32)
        zsum_ref[...] = jnp.zeros((N_FEAT, 1), jnp.float32)

    p = p_ref[...]
    pn_scale = lax.rsqrt(jnp.maximum(jnp.sum(p * p, axis=1, keepdims=True),
                                     jnp.float32(1e-24)))
    pn = p * pn_scale
    sims = lax.dot_general(qn_ref[...], pn, (((1,), (1,)), ((), ())),
                           preferred_element_type=jnp.float32)
    e = jnp.exp(sims * jnp.float32(INV_TEMP))
    zsum_ref[...] += jnp.sum(e, axis=1, keepdims=True)

    col = c * CHUNK + lax.broadcasted_iota(jnp.int32, (N_FEAT, CHUNK), 1)
    candv, candi = _extract_top8(e, col)
    mergev = jnp.concatenate([runv_ref[...], candv], axis=1)
    mergei = jnp.concatenate([runi_ref[...], candi], axis=1)
    runv_ref[...], runi_ref[...] = _extract_top8(mergev, mergei)

    @pl.when(c == N_CHUNKS - 1)
    def _fin():
        tv = runv_ref[...]
        s8 = jnp.sum(tv, axis=1, keepdims=True)
        denom = s8 + jnp.float32(1e-9) * zsum_ref[...]
        topv_ref[...] = tv / denom
        topi_ref[...] = runi_ref[...]


def _topk_tc(feat, prompts):
    return pl.pallas_call(
        _topk_tc_body,
        grid=(N_CHUNKS,),
        in_specs=[
            pl.BlockSpec((N_FEAT, DIM), lambda c: (0, 0)),
            pl.BlockSpec((CHUNK, DIM), lambda c: (c, 0)),
        ],
        out_specs=[
            pl.BlockSpec((N_FEAT, TOPK), lambda c: (0, 0)),
            pl.BlockSpec((N_FEAT, TOPK), lambda c: (0, 0)),
        ],
        out_shape=[
            jax.ShapeDtypeStruct((N_FEAT, TOPK), jnp.float32),
            jax.ShapeDtypeStruct((N_FEAT, TOPK), jnp.int32),
        ],
        scratch_shapes=[
            pltpu.VMEM((N_FEAT, DIM), jnp.float32),
            pltpu.VMEM((N_FEAT, TOPK), jnp.float32),
            pltpu.VMEM((N_FEAT, TOPK), jnp.int32),
            pltpu.VMEM((N_FEAT, 1), jnp.float32),
        ],
        compiler_params=pltpu.CompilerParams(
            dimension_semantics=("arbitrary",)),
    )(feat, prompts)


def _sc_body(topv_hbm, topi_hbm, prompts_hbm, probs_hbm, mixed_hbm,
             topv_v, topi_v, zbuf, idx_v, prow_v, mrow_v, sem):
    core = lax.axis_index("c")
    sub = lax.axis_index("s")
    wid = sub * 2 + core
    base = wid * ROWS_PER_W

    pltpu.sync_copy(topv_hbm.at[pl.ds(base, ROWS_PER_W)], topv_v)
    pltpu.sync_copy(topi_hbm.at[pl.ds(base, ROWS_PER_W)], topi_v)

    def zstep(i, carry):
        zbuf[pl.ds(i * 16, 16)] = jnp.zeros((16,), jnp.float32)
        return carry

    lax.fori_loop(0, N_CLASSES // 16, zstep, 0)

    lanes = lax.iota(jnp.int32, 16)
    k8 = lanes & 7
    m8 = lanes < 8
    zero16 = jnp.zeros((16,), jnp.float32)

    def row_step(r, carry):
        rr = jnp.zeros((16,), jnp.int32) + r
        idx16 = plsc.load_gather(topi_v, [rr, k8])
        v16 = plsc.load_gather(topv_v, [rr, k8])
        plsc.store_scatter(zbuf, [idx16], v16, mask=m8)
        idx_v[...] = idx16
        cp = pltpu.make_async_copy(prompts_hbm.at[idx_v], prow_v, sem)
        cp.start()
        cp.wait()
        ws = [plsc.load_gather(topv_v, [rr, jnp.full((16,), k, jnp.int32)])
              for k in range(TOPK)]
        for j in range(DIM // 16):
            acc = ws[0] * prow_v[0, pl.ds(j * 16, 16)]
            for k in range(1, TOPK):
                acc = acc + ws[k] * prow_v[k, pl.ds(j * 16, 16)]
            mrow_v[pl.ds(j * 16, 16)] = acc
        pltpu.sync_copy(mrow_v, mixed_hbm.at[base + r])
        pltpu.sync_copy(zbuf, probs_hbm.at[base + r])
        plsc.store_scatter(zbuf, [idx16], zero16, mask=m8)
        return carry

    lax.fori_loop(0, ROWS_PER_W, row_step, 0)


def _sc_scatter_mix(topv, topi, prompts):
    mesh = plsc.VectorSubcoreMesh(core_axis_name="c", subcore_axis_name="s")
    fn = pl.kernel(
        _sc_body,
        out_type=[
            jax.ShapeDtypeStruct((N_FEAT, N_CLASSES), jnp.float32),
            jax.ShapeDtypeStruct((N_FEAT, DIM), jnp.float32),
        ],
        mesh=mesh,
        scratch_types=[
            pltpu.VMEM((ROWS_PER_W, TOPK), jnp.float32),
            pltpu.VMEM((ROWS_PER_W, TOPK), jnp.int32),
            pltpu.VMEM((N_CLASSES,), jnp.float32),
            pltpu.VMEM((16,), jnp.int32),
            pltpu.VMEM((16, DIM), jnp.float32),
            pltpu.VMEM((DIM,), jnp.float32),
            pltpu.SemaphoreType.DMA,
        ],
    )
    return fn(topv, topi, prompts)


def kernel(feat, prompts, topk):
    del topk
    topv, topi = _topk_tc(feat, prompts)
    probs, mixed = _sc_scatter_mix(topv, topi, prompts)
    return (mixed, probs)


# speedup vs baseline: 2.4166x; 2.4166x over previous
import functools

import jax
import jax.numpy as jnp
from jax import lax
from jax.experimental import pallas as pl
from jax.experimental.pallas import tpu as pltpu
from jax.experimental.pallas import tpu_sc as plsc

N_FEAT = 1024
DIM = 128
N_CLASSES = 100000
CHUNK = 1000
N_CHUNKS = N_CLASSES // CHUNK
TOPK = 8
INV_TEMP = 1.0 / 0.07

NUM_WORKERS = 32
ROWS_PER_W = N_FEAT // NUM_WORKERS


def _extract_top8(vals, idxs):
    outv, outi = [], []
    big = jnp.int32(0x7FFFFFFF)
    for _ in range(TOPK):
        m = jnp.max(vals, axis=1, keepdims=True)
        im = jnp.min(jnp.where(vals == m, idxs, big), axis=1, keepdims=True)
        outv.append(m)
        outi.append(im)
        vals = jnp.where(idxs == im, jnp.float32(-2.0), vals)
    return jnp.concatenate(outv, axis=1), jnp.concatenate(outi, axis=1)


def _topk_tc_body(feat_ref, p_ref, topv_ref, topi_ref,
                  qn_ref, runv_ref, runi_ref, zsum_ref):
    c = pl.program_id(0)

    @pl.when(c == 0)
    def _init():
        q = feat_ref[...]
        n = jnp.sqrt(jnp.sum(q * q, axis=1, keepdims=True))
        qn_ref[...] = q / jnp.maximum(n, jnp.float32(1e-12))
        runv_ref[...] = jnp.full((N_FEAT, TOPK), -1.0, jnp.float32)
        runi_ref[...] = jnp.zeros((N_FEAT, TOPK), jnp.int32)
        zsum_ref[...] = jnp.zeros((N_FEAT, 1), jnp.float32)

    p = p_ref[...]
    pn_scale = lax.rsqrt(jnp.maximum(jnp.sum(p * p, axis=1, keepdims=True),
                                     jnp.float32(1e-24)))
    pn = p * pn_scale
    sims = lax.dot_general(qn_ref[...], pn, (((1,), (1,)), ((), ())),
                           preferred_element_type=jnp.float32)
    e = jnp.exp(sims * jnp.float32(INV_TEMP))
    zsum_ref[...] += jnp.sum(e, axis=1, keepdims=True)

    col = c * CHUNK + lax.broadcasted_iota(jnp.int32, (N_FEAT, CHUNK), 1)
    candv, candi = _extract_top8(e, col)
    mergev = jnp.concatenate([runv_ref[...], candv], axis=1)
    mergei = jnp.concatenate([runi_ref[...], candi], axis=1)
    runv_ref[...], runi_ref[...] = _extract_top8(mergev, mergei)

    @pl.when(c == N_CHUNKS - 1)
    def _fin():
        tv = runv_ref[...]
        s8 = jnp.sum(tv, axis=1, keepdims=True)
        denom = s8 + jnp.float32(1e-9) * zsum_ref[...]
        topv_ref[...] = tv / denom
        topi_ref[...] = runi_ref[...]


def _topk_tc(feat, prompts):
    return pl.pallas_call(
        _topk_tc_body,
        grid=(N_CHUNKS,),
        in_specs=[
            pl.BlockSpec((N_FEAT, DIM), lambda c: (0, 0)),
            pl.BlockSpec((CHUNK, DIM), lambda c: (c, 0)),
        ],
        out_specs=[
            pl.BlockSpec((N_FEAT, TOPK), lambda c: (0, 0)),
            pl.BlockSpec((N_FEAT, TOPK), lambda c: (0, 0)),
        ],
        out_shape=[
            jax.ShapeDtypeStruct((N_FEAT, TOPK), jnp.float32),
            jax.ShapeDtypeStruct((N_FEAT, TOPK), jnp.int32),
        ],
        scratch_shapes=[
            pltpu.VMEM((N_FEAT, DIM), jnp.float32),
            pltpu.VMEM((N_FEAT, TOPK), jnp.float32),
            pltpu.VMEM((N_FEAT, TOPK), jnp.int32),
            pltpu.VMEM((N_FEAT, 1), jnp.float32),
        ],
        compiler_params=pltpu.CompilerParams(
            dimension_semantics=("arbitrary",)),
    )(feat, prompts)


def _sc_body(topv_hbm, topi_hbm, prompts_hbm, probs_hbm, mixed_hbm,
             topv_v, topi_v, zbuf, idx_v, prow_v, mrow_v, sem):
    core = lax.axis_index("c")
    sub = lax.axis_index("s")
    wid = sub * 2 + core
    base = wid * ROWS_PER_W

    pltpu.sync_copy(topv_hbm.at[pl.ds(base * TOPK, ROWS_PER_W * TOPK)], topv_v)
    pltpu.sync_copy(topi_hbm.at[pl.ds(base * TOPK, ROWS_PER_W * TOPK)], topi_v)

    def zstep(i, carry):
        zbuf[pl.ds(i * 16, 16)] = jnp.zeros((16,), jnp.float32)
        return carry

    lax.fori_loop(0, N_CLASSES // 16, zstep, 0)

    lanes = lax.iota(jnp.int32, 16)
    k8 = lanes & 7
    m8 = lanes < 8
    zero16 = jnp.zeros((16,), jnp.float32)

    def row_step(r, carry):
        rbase = jnp.zeros((16,), jnp.int32) + r * TOPK
        idx16 = plsc.load_gather(topi_v, [rbase + k8])
        v16 = plsc.load_gather(topv_v, [rbase + k8])
        plsc.store_scatter(zbuf, [idx16], v16, mask=m8)
        idx_v[...] = idx16
        cp = pltpu.make_async_copy(prompts_hbm.at[idx_v], prow_v, sem)
        cp.start()
        cp.wait()
        ws = [plsc.load_gather(topv_v, [rbase + k])
              for k in range(TOPK)]
        for j in range(DIM // 16):
            acc = ws[0] * prow_v[0, pl.ds(j * 16, 16)]
            for k in range(1, TOPK):
                acc = acc + ws[k] * prow_v[k, pl.ds(j * 16, 16)]
            mrow_v[pl.ds(j * 16, 16)] = acc
        pltpu.sync_copy(mrow_v, mixed_hbm.at[base + r])
        pltpu.sync_copy(zbuf, probs_hbm.at[base + r])
        plsc.store_scatter(zbuf, [idx16], zero16, mask=m8)
        return carry

    lax.fori_loop(0, ROWS_PER_W, row_step, 0)


def _sc_scatter_mix(topv, topi, prompts):
    mesh = plsc.VectorSubcoreMesh(core_axis_name="c", subcore_axis_name="s")
    fn = pl.kernel(
        _sc_body,
        out_type=[
            jax.ShapeDtypeStruct((N_FEAT, N_CLASSES), jnp.float32),
            jax.ShapeDtypeStruct((N_FEAT, DIM), jnp.float32),
        ],
        mesh=mesh,
        scratch_types=[
            pltpu.VMEM((ROWS_PER_W * TOPK,), jnp.float32),
            pltpu.VMEM((ROWS_PER_W * TOPK,), jnp.int32),
            pltpu.VMEM((N_CLASSES,), jnp.float32),
            pltpu.VMEM((16,), jnp.int32),
            pltpu.VMEM((16, DIM), jnp.float32),
            pltpu.VMEM((DIM,), jnp.float32),
            pltpu.SemaphoreType.DMA,
        ],
        compiler_params=pltpu.CompilerParams(needs_layout_passes=False),
    )
    return fn(topv.reshape(-1), topi.reshape(-1), prompts)


def kernel(feat, prompts, topk):
    del topk
    topv, topi = _topk_tc(feat, prompts)
    probs, mixed = _sc_scatter_mix(topv, topi, prompts)
    return (mixed, probs)


# speedup vs baseline: 2.9624x; 1.2258x over previous
import functools

import jax
import jax.numpy as jnp
from jax import lax
from jax.experimental import pallas as pl
from jax.experimental.pallas import tpu as pltpu
from jax.experimental.pallas import tpu_sc as plsc

N_FEAT = 1024
DIM = 128
N_CLASSES = 100000
CHUNK = 1000
N_CHUNKS = N_CLASSES // CHUNK
TOPK = 8
INV_TEMP = 1.0 / 0.07

NUM_WORKERS = 32
ROWS_PER_W = N_FEAT // NUM_WORKERS


def _extract_top8(vals, idxs):
    outv, outi = [], []
    big = jnp.float32(3.0e7)
    for _ in range(TOPK):
        m = jnp.max(vals, axis=1, keepdims=True)
        im = jnp.min(jnp.where(vals == m, idxs, big), axis=1, keepdims=True)
        outv.append(m)
        outi.append(im)
        vals = jnp.where(idxs == im, jnp.float32(-2.0), vals)
    return jnp.concatenate(outv, axis=1), jnp.concatenate(outi, axis=1)


def _topk_tc_body(feat_ref, p_ref, topv_ref, topi_ref,
                  qn_ref, runv_ref, runi_ref, zsum_ref):
    c = pl.program_id(0)

    @pl.when(c == 0)
    def _init():
        q = feat_ref[...]
        n = jnp.sqrt(jnp.sum(q * q, axis=1, keepdims=True))
        qn_ref[...] = q / jnp.maximum(n, jnp.float32(1e-12))
        runv_ref[...] = jnp.full((N_FEAT, TOPK), -1.0, jnp.float32)
        runi_ref[...] = jnp.zeros((N_FEAT, TOPK), jnp.float32)
        zsum_ref[...] = jnp.zeros((N_FEAT, 1), jnp.float32)

    p = p_ref[...]
    pn_scale = lax.rsqrt(jnp.maximum(jnp.sum(p * p, axis=1, keepdims=True),
                                     jnp.float32(1e-24)))
    pn = p * pn_scale
    sims = lax.dot_general(qn_ref[...], pn, (((1,), (1,)), ((), ())),
                           preferred_element_type=jnp.float32)
    e = jnp.exp(sims * jnp.float32(INV_TEMP))
    zsum_ref[...] += jnp.sum(e, axis=1, keepdims=True)

    col = ((c * CHUNK).astype(jnp.float32)
           + lax.broadcasted_iota(jnp.int32, (N_FEAT, CHUNK), 1
                                  ).astype(jnp.float32))
    candv, candi = _extract_top8(e, col)
    mergev = jnp.concatenate([runv_ref[...], candv], axis=1)
    mergei = jnp.concatenate([runi_ref[...], candi], axis=1)
    runv_ref[...], runi_ref[...] = _extract_top8(mergev, mergei)

    @pl.when(c == N_CHUNKS - 1)
    def _fin():
        tv = runv_ref[...]
        s8 = jnp.sum(tv, axis=1, keepdims=True)
        denom = s8 + jnp.float32(1e-9) * zsum_ref[...]
        topv_ref[...] = tv / denom
        topi_ref[...] = runi_ref[...].astype(jnp.int32)


def _topk_tc(feat, prompts):
    return pl.pallas_call(
        _topk_tc_body,
        grid=(N_CHUNKS,),
        in_specs=[
            pl.BlockSpec((N_FEAT, DIM), lambda c: (0, 0)),
            pl.BlockSpec((CHUNK, DIM), lambda c: (c, 0)),
        ],
        out_specs=[
            pl.BlockSpec((N_FEAT, TOPK), lambda c: (0, 0)),
            pl.BlockSpec((N_FEAT, TOPK), lambda c: (0, 0)),
        ],
        out_shape=[
            jax.ShapeDtypeStruct((N_FEAT, TOPK), jnp.float32),
            jax.ShapeDtypeStruct((N_FEAT, TOPK), jnp.int32),
        ],
        scratch_shapes=[
            pltpu.VMEM((N_FEAT, DIM), jnp.float32),
            pltpu.VMEM((N_FEAT, TOPK), jnp.float32),
            pltpu.VMEM((N_FEAT, TOPK), jnp.float32),
            pltpu.VMEM((N_FEAT, 1), jnp.float32),
        ],
        compiler_params=pltpu.CompilerParams(
            dimension_semantics=("arbitrary",)),
    )(feat, prompts)


def _sc_body(topv_hbm, topi_hbm, prompts_hbm, probs_hbm, mixed_hbm,
             topv_v, topi_v, zbuf, idx_v, prow_v, mrow_v, sem):
    core = lax.axis_index("c")
    sub = lax.axis_index("s")
    wid = sub * 2 + core
    base = wid * ROWS_PER_W

    pltpu.sync_copy(topv_hbm.at[pl.ds(base * TOPK, ROWS_PER_W * TOPK)], topv_v)
    pltpu.sync_copy(topi_hbm.at[pl.ds(base * TOPK, ROWS_PER_W * TOPK)], topi_v)

    def zstep(i, carry):
        zbuf[pl.ds(i * 16, 16)] = jnp.zeros((16,), jnp.float32)
        return carry

    lax.fori_loop(0, N_CLASSES // 16, zstep, 0)

    lanes = lax.iota(jnp.int32, 16)
    k8 = lanes & 7
    m8 = lanes < 8
    zero16 = jnp.zeros((16,), jnp.float32)

    def row_step(r, carry):
        rbase = jnp.zeros((16,), jnp.int32) + r * TOPK
        idx16 = plsc.load_gather(topi_v, [rbase + k8])
        v16 = plsc.load_gather(topv_v, [rbase + k8])
        plsc.store_scatter(zbuf, [idx16], v16, mask=m8)
        idx_v[...] = idx16
        cp = pltpu.make_async_copy(prompts_hbm.at[idx_v], prow_v, sem)
        cp.start()
        cp.wait()
        ws = [plsc.load_gather(topv_v, [rbase + k])
              for k in range(TOPK)]
        for j in range(DIM // 16):
            acc = ws[0] * prow_v[0, pl.ds(j * 16, 16)]
            for k in range(1, TOPK):
                acc = acc + ws[k] * prow_v[k, pl.ds(j * 16, 16)]
            mrow_v[pl.ds(j * 16, 16)] = acc
        pltpu.sync_copy(mrow_v, mixed_hbm.at[base + r])
        pltpu.sync_copy(zbuf, probs_hbm.at[base + r])
        plsc.store_scatter(zbuf, [idx16], zero16, mask=m8)
        return carry

    lax.fori_loop(0, ROWS_PER_W, row_step, 0)


def _sc_scatter_mix(topv, topi, prompts):
    mesh = plsc.VectorSubcoreMesh(core_axis_name="c", subcore_axis_name="s")
    fn = pl.kernel(
        _sc_body,
        out_type=[
            jax.ShapeDtypeStruct((N_FEAT, N_CLASSES), jnp.float32),
            jax.ShapeDtypeStruct((N_FEAT, DIM), jnp.float32),
        ],
        mesh=mesh,
        scratch_types=[
            pltpu.VMEM((ROWS_PER_W * TOPK,), jnp.float32),
            pltpu.VMEM((ROWS_PER_W * TOPK,), jnp.int32),
            pltpu.VMEM((N_CLASSES,), jnp.float32),
            pltpu.VMEM((16,), jnp.int32),
            pltpu.VMEM((16, DIM), jnp.float32),
            pltpu.VMEM((DIM,), jnp.float32),
            pltpu.SemaphoreType.DMA,
        ],
        compiler_params=pltpu.CompilerParams(needs_layout_passes=False),
    )
    return fn(topv.reshape(-1), topi.reshape(-1), prompts)


def kernel(feat, prompts, topk):
    del topk
    topv, topi = _topk_tc(feat, prompts)
    probs, mixed = _sc_scatter_mix(topv, topi, prompts)
    return (mixed, probs)


# speedup vs baseline: 3.2857x; 1.1091x over previous
import functools

import jax
import jax.numpy as jnp
from jax import lax
from jax.experimental import pallas as pl
from jax.experimental.pallas import tpu as pltpu
from jax.experimental.pallas import tpu_sc as plsc

N_FEAT = 1024
DIM = 128
N_CLASSES = 100000
CHUNK = 1000
N_CHUNKS = N_CLASSES // CHUNK
TOPK = 8
INV_TEMP = 1.0 / 0.07

NUM_WORKERS = 32
ROWS_PER_W = N_FEAT // NUM_WORKERS


def _extract_top8(vals, idxs):
    outv, outi = [], []
    big = jnp.float32(3.0e7)
    for _ in range(TOPK):
        m = jnp.max(vals, axis=1, keepdims=True)
        im = jnp.min(jnp.where(vals == m, idxs, big), axis=1, keepdims=True)
        outv.append(m)
        outi.append(im)
        vals = jnp.where(idxs == im, jnp.float32(-2.0), vals)
    return jnp.concatenate(outv, axis=1), jnp.concatenate(outi, axis=1)


MERGE_EVERY = 4


def _topk_tc_body(feat_ref, p_ref, topv_ref, topi_ref,
                  qn_ref, runv_ref, runi_ref, zsum_ref,
                  candv_ref, candi_ref):
    c = pl.program_id(0)

    @pl.when(c == 0)
    def _init():
        q = feat_ref[...]
        n = jnp.sqrt(jnp.sum(q * q, axis=1, keepdims=True))
        qn_ref[...] = q / jnp.maximum(n, jnp.float32(1e-12))
        runv_ref[...] = jnp.full((N_FEAT, TOPK), -1.0, jnp.float32)
        runi_ref[...] = jnp.zeros((N_FEAT, TOPK), jnp.float32)
        zsum_ref[...] = jnp.zeros((N_FEAT, 1), jnp.float32)

    p = p_ref[...]
    pn_scale = lax.rsqrt(jnp.maximum(jnp.sum(p * p, axis=1, keepdims=True),
                                     jnp.float32(1e-24)))
    pn = p * pn_scale
    sims = lax.dot_general(qn_ref[...], pn, (((1,), (1,)), ((), ())),
                           preferred_element_type=jnp.float32)
    e = jnp.exp(sims * jnp.float32(INV_TEMP))
    zsum_ref[...] += jnp.sum(e, axis=1, keepdims=True)

    col = ((c * CHUNK).astype(jnp.float32)
           + lax.broadcasted_iota(jnp.int32, (N_FEAT, CHUNK), 1
                                  ).astype(jnp.float32))
    candv, candi = _extract_top8(e, col)
    slot = lax.rem(c, MERGE_EVERY)
    for b in range(MERGE_EVERY):
        @pl.when(slot == b)
        def _store(b=b):
            candv_ref[:, b * TOPK:(b + 1) * TOPK] = candv
            candi_ref[:, b * TOPK:(b + 1) * TOPK] = candi

    @pl.when(slot == MERGE_EVERY - 1)
    def _merge():
        mergev = jnp.concatenate([runv_ref[...], candv_ref[...]], axis=1)
        mergei = jnp.concatenate([runi_ref[...], candi_ref[...]], axis=1)
        mv, mi = _extract_top8(mergev, mergei)
        runv_ref[...] = mv
        runi_ref[...] = mi

    @pl.when(c == N_CHUNKS - 1)
    def _fin():
        tv = runv_ref[...]
        s8 = jnp.sum(tv, axis=1, keepdims=True)
        denom = s8 + jnp.float32(1e-9) * zsum_ref[...]
        topv_ref[...] = tv / denom
        topi_ref[...] = runi_ref[...].astype(jnp.int32)


def _topk_tc(feat, prompts):
    return pl.pallas_call(
        _topk_tc_body,
        grid=(N_CHUNKS,),
        in_specs=[
            pl.BlockSpec((N_FEAT, DIM), lambda c: (0, 0)),
            pl.BlockSpec((CHUNK, DIM), lambda c: (c, 0)),
        ],
        out_specs=[
            pl.BlockSpec((N_FEAT, TOPK), lambda c: (0, 0)),
            pl.BlockSpec((N_FEAT, TOPK), lambda c: (0, 0)),
        ],
        out_shape=[
            jax.ShapeDtypeStruct((N_FEAT, TOPK), jnp.float32),
            jax.ShapeDtypeStruct((N_FEAT, TOPK), jnp.int32),
        ],
        scratch_shapes=[
            pltpu.VMEM((N_FEAT, DIM), jnp.float32),
            pltpu.VMEM((N_FEAT, TOPK), jnp.float32),
            pltpu.VMEM((N_FEAT, TOPK), jnp.float32),
            pltpu.VMEM((N_FEAT, 1), jnp.float32),
            pltpu.VMEM((N_FEAT, MERGE_EVERY * TOPK), jnp.float32),
            pltpu.VMEM((N_FEAT, MERGE_EVERY * TOPK), jnp.float32),
        ],
        compiler_params=pltpu.CompilerParams(
            dimension_semantics=("arbitrary",)),
    )(feat, prompts)


def _sc_body(topv_hbm, topi_hbm, prompts_hbm, probs_hbm, mixed_hbm,
             topv_v, topi_v, zbuf, idx_v, prow_v, mrow_v, sem, sem2):
    core = lax.axis_index("c")
    sub = lax.axis_index("s")
    wid = sub * 2 + core
    base = wid * ROWS_PER_W

    pltpu.sync_copy(topv_hbm.at[pl.ds(base * TOPK, ROWS_PER_W * TOPK)], topv_v)
    pltpu.sync_copy(topi_hbm.at[pl.ds(base * TOPK, ROWS_PER_W * TOPK)], topi_v)

    def zstep(i, carry):
        zbuf[pl.ds(i * 16, 16)] = jnp.zeros((16,), jnp.float32)
        return carry

    lax.fori_loop(0, N_CLASSES // 16, zstep, 0)

    lanes = lax.iota(jnp.int32, 16)
    k8 = lanes & 7
    m8 = lanes < 8
    zero16 = jnp.zeros((16,), jnp.float32)

    def row_step(r, carry):
        rbase = jnp.zeros((16,), jnp.int32) + r * TOPK
        idx16 = plsc.load_gather(topi_v, [rbase + k8])
        v16 = plsc.load_gather(topv_v, [rbase + k8])
        plsc.store_scatter(zbuf, [idx16], v16, mask=m8)
        cpz = pltpu.make_async_copy(zbuf, probs_hbm.at[base + r], sem2)
        cpz.start()
        idx_v[...] = idx16
        cp = pltpu.make_async_copy(prompts_hbm.at[idx_v], prow_v, sem)
        cp.start()
        cp.wait()
        ws = [plsc.load_gather(topv_v, [rbase + k])
              for k in range(TOPK)]
        for j in range(DIM // 16):
            acc = ws[0] * prow_v[0, pl.ds(j * 16, 16)]
            for k in range(1, TOPK):
                acc = acc + ws[k] * prow_v[k, pl.ds(j * 16, 16)]
            mrow_v[pl.ds(j * 16, 16)] = acc
        pltpu.sync_copy(mrow_v, mixed_hbm.at[base + r])
        cpz.wait()
        plsc.store_scatter(zbuf, [idx16], zero16, mask=m8)
        return carry

    lax.fori_loop(0, ROWS_PER_W, row_step, 0)


def _sc_scatter_mix(topv, topi, prompts):
    mesh = plsc.VectorSubcoreMesh(core_axis_name="c", subcore_axis_name="s")
    fn = pl.kernel(
        _sc_body,
        out_type=[
            jax.ShapeDtypeStruct((N_FEAT, N_CLASSES), jnp.float32),
            jax.ShapeDtypeStruct((N_FEAT, DIM), jnp.float32),
        ],
        mesh=mesh,
        scratch_types=[
            pltpu.VMEM((ROWS_PER_W * TOPK,), jnp.float32),
            pltpu.VMEM((ROWS_PER_W * TOPK,), jnp.int32),
            pltpu.VMEM((N_CLASSES,), jnp.float32),
            pltpu.VMEM((16,), jnp.int32),
            pltpu.VMEM((16, DIM), jnp.float32),
            pltpu.VMEM((DIM,), jnp.float32),
            pltpu.SemaphoreType.DMA,
            pltpu.SemaphoreType.DMA,
        ],
        compiler_params=pltpu.CompilerParams(needs_layout_passes=False),
    )
    return fn(topv.reshape(-1), topi.reshape(-1), prompts)


def kernel(feat, prompts, topk):
    del topk
    topv, topi = _topk_tc(feat, prompts)
    probs, mixed = _sc_scatter_mix(topv, topi, prompts)
    return (mixed, probs)


# speedup vs baseline: 3.3554x; 1.0212x over previous
import functools

import jax
import jax.numpy as jnp
from jax import lax
from jax.experimental import pallas as pl
from jax.experimental.pallas import tpu as pltpu
from jax.experimental.pallas import tpu_sc as plsc

N_FEAT = 1024
DIM = 128
N_CLASSES = 100000
CHUNK = 1000
N_CHUNKS = N_CLASSES // CHUNK
TOPK = 8
INV_TEMP = 1.0 / 0.07

NUM_WORKERS = 32
ROWS_PER_W = N_FEAT // NUM_WORKERS


def _extract_top8_to(vals, idxs, outv_ref, outi_ref):
    big = jnp.float32(3.0e7)
    for k in range(TOPK):
        m = jnp.max(vals, axis=1, keepdims=True)
        im = jnp.min(jnp.where(vals == m, idxs, big), axis=1, keepdims=True)
        outv_ref[:, k:k + 1] = m
        outi_ref[:, k:k + 1] = im
        if k < TOPK - 1:
            vals = jnp.where(idxs == im, jnp.float32(-2.0), vals)


MERGE_EVERY = 4


def _topk_tc_body(feat_ref, p_ref, topv_ref, topi_ref,
                  qn_ref, runv_ref, runi_ref, zacc_ref,
                  candv_ref, candi_ref, ctmp_v, ctmp_i):
    c = pl.program_id(0)

    @pl.when(c == 0)
    def _init():
        q = feat_ref[...]
        n = jnp.sqrt(jnp.sum(q * q, axis=1, keepdims=True))
        qn_ref[...] = q / jnp.maximum(n, jnp.float32(1e-12))
        runv_ref[...] = jnp.full((N_FEAT, TOPK), -1.0, jnp.float32)
        runi_ref[...] = jnp.zeros((N_FEAT, TOPK), jnp.float32)
        zacc_ref[...] = jnp.zeros((N_FEAT, CHUNK), jnp.float32)

    p = p_ref[...]
    pn_scale = lax.rsqrt(jnp.maximum(jnp.sum(p * p, axis=1, keepdims=True),
                                     jnp.float32(1e-24)))
    pn = p * pn_scale
    sims = lax.dot_general(qn_ref[...], pn, (((1,), (1,)), ((), ())),
                           preferred_element_type=jnp.float32)
    e = jnp.exp(sims * jnp.float32(INV_TEMP))
    zacc_ref[...] += e

    col = ((c * CHUNK).astype(jnp.float32)
           + lax.broadcasted_iota(jnp.int32, (N_FEAT, CHUNK), 1
                                  ).astype(jnp.float32))
    _extract_top8_to(e, col, ctmp_v, ctmp_i)
    slot = lax.rem(c, MERGE_EVERY)
    for b in range(MERGE_EVERY):
        @pl.when(slot == b)
        def _store(b=b):
            candv_ref[:, b * TOPK:(b + 1) * TOPK] = ctmp_v[...]
            candi_ref[:, b * TOPK:(b + 1) * TOPK] = ctmp_i[...]

    @pl.when(slot == MERGE_EVERY - 1)
    def _merge():
        mergev = jnp.concatenate([runv_ref[...], candv_ref[...]], axis=1)
        mergei = jnp.concatenate([runi_ref[...], candi_ref[...]], axis=1)
        _extract_top8_to(mergev, mergei, runv_ref, runi_ref)

    @pl.when(c == N_CHUNKS - 1)
    def _fin():
        tv = runv_ref[...]
        s8 = jnp.sum(tv, axis=1, keepdims=True)
        zsum = jnp.sum(zacc_ref[...], axis=1, keepdims=True)
        denom = s8 + jnp.float32(1e-9) * zsum
        topv_ref[...] = tv / denom
        topi_ref[...] = runi_ref[...].astype(jnp.int32)


def _topk_tc(feat, prompts):
    return pl.pallas_call(
        _topk_tc_body,
        grid=(N_CHUNKS,),
        in_specs=[
            pl.BlockSpec((N_FEAT, DIM), lambda c: (0, 0)),
            pl.BlockSpec((CHUNK, DIM), lambda c: (c, 0)),
        ],
        out_specs=[
            pl.BlockSpec((N_FEAT, TOPK), lambda c: (0, 0)),
            pl.BlockSpec((N_FEAT, TOPK), lambda c: (0, 0)),
        ],
        out_shape=[
            jax.ShapeDtypeStruct((N_FEAT, TOPK), jnp.float32),
            jax.ShapeDtypeStruct((N_FEAT, TOPK), jnp.int32),
        ],
        scratch_shapes=[
            pltpu.VMEM((N_FEAT, DIM), jnp.float32),
            pltpu.VMEM((N_FEAT, TOPK), jnp.float32),
            pltpu.VMEM((N_FEAT, TOPK), jnp.float32),
            pltpu.VMEM((N_FEAT, CHUNK), jnp.float32),
            pltpu.VMEM((N_FEAT, MERGE_EVERY * TOPK), jnp.float32),
            pltpu.VMEM((N_FEAT, MERGE_EVERY * TOPK), jnp.float32),
            pltpu.VMEM((N_FEAT, TOPK), jnp.float32),
            pltpu.VMEM((N_FEAT, TOPK), jnp.float32),
        ],
        compiler_params=pltpu.CompilerParams(
            dimension_semantics=("arbitrary",)),
    )(feat, prompts)


def _sc_body(topv_hbm, topi_hbm, prompts_hbm, probs_hbm, mixed_hbm,
             topv_v, topi_v, zbuf, idx_v, prow_v, mrow_v, sem, sem2):
    core = lax.axis_index("c")
    sub = lax.axis_index("s")
    wid = sub * 2 + core
    base = wid * ROWS_PER_W

    pltpu.sync_copy(topv_hbm.at[pl.ds(base * TOPK, ROWS_PER_W * TOPK)], topv_v)
    pltpu.sync_copy(topi_hbm.at[pl.ds(base * TOPK, ROWS_PER_W * TOPK)], topi_v)

    def zstep(i, carry):
        zbuf[pl.ds(i * 16, 16)] = jnp.zeros((16,), jnp.float32)
        return carry

    lax.fori_loop(0, N_CLASSES // 16, zstep, 0)

    lanes = lax.iota(jnp.int32, 16)
    k8 = lanes & 7
    m8 = lanes < 8
    zero16 = jnp.zeros((16,), jnp.float32)

    def row_step(r, carry):
        rbase = jnp.zeros((16,), jnp.int32) + r * TOPK
        idx16 = plsc.load_gather(topi_v, [rbase + k8])
        v16 = plsc.load_gather(topv_v, [rbase + k8])
        plsc.store_scatter(zbuf, [idx16], v16, mask=m8)
        cpz = pltpu.make_async_copy(zbuf, probs_hbm.at[base + r], sem2)
        cpz.start()
        idx_v[...] = idx16
        cp = pltpu.make_async_copy(prompts_hbm.at[idx_v], prow_v, sem)
        cp.start()
        cp.wait()
        ws = [plsc.load_gather(topv_v, [rbase + k])
              for k in range(TOPK)]
        for j in range(DIM // 16):
            acc = ws[0] * prow_v[0, pl.ds(j * 16, 16)]
            for k in range(1, TOPK):
                acc = acc + ws[k] * prow_v[k, pl.ds(j * 16, 16)]
            mrow_v[pl.ds(j * 16, 16)] = acc
        pltpu.sync_copy(mrow_v, mixed_hbm.at[base + r])
        cpz.wait()
        plsc.store_scatter(zbuf, [idx16], zero16, mask=m8)
        return carry

    lax.fori_loop(0, ROWS_PER_W, row_step, 0)


def _sc_scatter_mix(topv, topi, prompts):
    mesh = plsc.VectorSubcoreMesh(core_axis_name="c", subcore_axis_name="s")
    fn = pl.kernel(
        _sc_body,
        out_type=[
            jax.ShapeDtypeStruct((N_FEAT, N_CLASSES), jnp.float32),
            jax.ShapeDtypeStruct((N_FEAT, DIM), jnp.float32),
        ],
        mesh=mesh,
        scratch_types=[
            pltpu.VMEM((ROWS_PER_W * TOPK,), jnp.float32),
            pltpu.VMEM((ROWS_PER_W * TOPK,), jnp.int32),
            pltpu.VMEM((N_CLASSES,), jnp.float32),
            pltpu.VMEM((16,), jnp.int32),
            pltpu.VMEM((16, DIM), jnp.float32),
            pltpu.VMEM((DIM,), jnp.float32),
            pltpu.SemaphoreType.DMA,
            pltpu.SemaphoreType.DMA,
        ],
        compiler_params=pltpu.CompilerParams(needs_layout_passes=False),
    )
    return fn(topv.reshape(-1), topi.reshape(-1), prompts)


def kernel(feat, prompts, topk):
    del topk
    topv, topi = _topk_tc(feat, prompts)
    probs, mixed = _sc_scatter_mix(topv, topi, prompts)
    return (mixed, probs)


# speedup vs baseline: 3.4662x; 1.0330x over previous
import functools

import jax
import jax.numpy as jnp
from jax import lax
from jax.experimental import pallas as pl
from jax.experimental.pallas import tpu as pltpu
from jax.experimental.pallas import tpu_sc as plsc

N_FEAT = 1024
DIM = 128
N_CLASSES = 100000
CHUNK = 1000
N_CHUNKS = N_CLASSES // CHUNK
TOPK = 8
INV_TEMP = 1.0 / 0.07

NUM_WORKERS = 32
ROWS_PER_W = N_FEAT // NUM_WORKERS


def _extract_top8_to(vals, idxs, outv_ref, outi_ref):
    big = jnp.float32(3.0e7)
    for k in range(TOPK):
        m = jnp.max(vals, axis=1, keepdims=True)
        im = jnp.min(jnp.where(vals == m, idxs, big), axis=1, keepdims=True)
        outv_ref[:, k:k + 1] = m
        outi_ref[:, k:k + 1] = im
        if k < TOPK - 1:
            vals = jnp.where(idxs == im, jnp.float32(-2.0), vals)


MERGE_EVERY = 10


def _topk_tc_body(feat_ref, p_ref, topv_ref, topi_ref,
                  qn_ref, runv_ref, runi_ref, zacc_ref,
                  candv_ref, candi_ref, ctmp_v, ctmp_i):
    c = pl.program_id(0)

    @pl.when(c == 0)
    def _init():
        q = feat_ref[...]
        n = jnp.sqrt(jnp.sum(q * q, axis=1, keepdims=True))
        qn_ref[...] = q / jnp.maximum(n, jnp.float32(1e-12))
        runv_ref[...] = jnp.full((N_FEAT, TOPK), -1.0, jnp.float32)
        runi_ref[...] = jnp.zeros((N_FEAT, TOPK), jnp.float32)
        zacc_ref[...] = jnp.zeros((N_FEAT, CHUNK), jnp.float32)

    p = p_ref[...]
    pn_scale = lax.rsqrt(jnp.maximum(jnp.sum(p * p, axis=1, keepdims=True),
                                     jnp.float32(1e-24)))
    pn = p * pn_scale
    sims = lax.dot_general(qn_ref[...], pn, (((1,), (1,)), ((), ())),
                           preferred_element_type=jnp.float32)
    e = jnp.exp(sims * jnp.float32(INV_TEMP))
    zacc_ref[...] += e

    col = ((c * CHUNK).astype(jnp.float32)
           + lax.broadcasted_iota(jnp.int32, (N_FEAT, CHUNK), 1
                                  ).astype(jnp.float32))
    _extract_top8_to(e, col, ctmp_v, ctmp_i)
    slot = lax.rem(c, MERGE_EVERY)
    for b in range(MERGE_EVERY):
        @pl.when(slot == b)
        def _store(b=b):
            candv_ref[:, b * TOPK:(b + 1) * TOPK] = ctmp_v[...]
            candi_ref[:, b * TOPK:(b + 1) * TOPK] = ctmp_i[...]

    @pl.when(slot == MERGE_EVERY - 1)
    def _merge():
        mergev = jnp.concatenate([runv_ref[...], candv_ref[...]], axis=1)
        mergei = jnp.concatenate([runi_ref[...], candi_ref[...]], axis=1)
        _extract_top8_to(mergev, mergei, runv_ref, runi_ref)

    @pl.when(c == N_CHUNKS - 1)
    def _fin():
        tv = runv_ref[...]
        s8 = jnp.sum(tv, axis=1, keepdims=True)
        zsum = jnp.sum(zacc_ref[...], axis=1, keepdims=True)
        denom = s8 + jnp.float32(1e-9) * zsum
        topv_ref[...] = tv / denom
        topi_ref[...] = runi_ref[...].astype(jnp.int32)


def _topk_tc(feat, prompts):
    return pl.pallas_call(
        _topk_tc_body,
        grid=(N_CHUNKS,),
        in_specs=[
            pl.BlockSpec((N_FEAT, DIM), lambda c: (0, 0)),
            pl.BlockSpec((CHUNK, DIM), lambda c: (c, 0)),
        ],
        out_specs=[
            pl.BlockSpec((N_FEAT, TOPK), lambda c: (0, 0)),
            pl.BlockSpec((N_FEAT, TOPK), lambda c: (0, 0)),
        ],
        out_shape=[
            jax.ShapeDtypeStruct((N_FEAT, TOPK), jnp.float32),
            jax.ShapeDtypeStruct((N_FEAT, TOPK), jnp.int32),
        ],
        scratch_shapes=[
            pltpu.VMEM((N_FEAT, DIM), jnp.float32),
            pltpu.VMEM((N_FEAT, TOPK), jnp.float32),
            pltpu.VMEM((N_FEAT, TOPK), jnp.float32),
            pltpu.VMEM((N_FEAT, CHUNK), jnp.float32),
            pltpu.VMEM((N_FEAT, MERGE_EVERY * TOPK), jnp.float32),
            pltpu.VMEM((N_FEAT, MERGE_EVERY * TOPK), jnp.float32),
            pltpu.VMEM((N_FEAT, TOPK), jnp.float32),
            pltpu.VMEM((N_FEAT, TOPK), jnp.float32),
        ],
        compiler_params=pltpu.CompilerParams(
            dimension_semantics=("arbitrary",)),
    )(feat, prompts)


def _sc_body(topv_hbm, topi_hbm, prompts_hbm, probs_hbm, mixed_hbm,
             topv_v, topi_v, zbuf, idx_v, prow_v, mrow_v, sem, sem2):
    core = lax.axis_index("c")
    sub = lax.axis_index("s")
    wid = sub * 2 + core
    base = wid * ROWS_PER_W

    pltpu.sync_copy(topv_hbm.at[pl.ds(base * TOPK, ROWS_PER_W * TOPK)], topv_v)
    pltpu.sync_copy(topi_hbm.at[pl.ds(base * TOPK, ROWS_PER_W * TOPK)], topi_v)

    def zstep(i, carry):
        z16 = jnp.zeros((16,), jnp.float32)
        zbuf[pl.ds(i * 64, 16)] = z16
        zbuf[pl.ds(i * 64 + 16, 16)] = z16
        zbuf[pl.ds(i * 64 + 32, 16)] = z16
        zbuf[pl.ds(i * 64 + 48, 16)] = z16
        return carry

    lax.fori_loop(0, N_CLASSES // 64, zstep, 0)
    for t in range((N_CLASSES % 64) // 16):
        zbuf[pl.ds((N_CLASSES // 64) * 64 + t * 16, 16)] = (
            jnp.zeros((16,), jnp.float32))

    lanes = lax.iota(jnp.int32, 16)
    k8 = lanes & 7
    m8 = lanes < 8
    zero16 = jnp.zeros((16,), jnp.float32)

    def row_step(r, carry):
        rbase = jnp.zeros((16,), jnp.int32) + r * TOPK
        idx16 = plsc.load_gather(topi_v, [rbase + k8])
        v16 = plsc.load_gather(topv_v, [rbase + k8])
        plsc.store_scatter(zbuf, [idx16], v16, mask=m8)
        cpz = pltpu.make_async_copy(zbuf, probs_hbm.at[base + r], sem2)
        cpz.start()
        idx_v[...] = idx16
        cp = pltpu.make_async_copy(prompts_hbm.at[idx_v], prow_v, sem)
        cp.start()
        cp.wait()
        ws = [plsc.load_gather(topv_v, [rbase + k])
              for k in range(TOPK)]
        for j in range(DIM // 16):
            acc = ws[0] * prow_v[0, pl.ds(j * 16, 16)]
            for k in range(1, TOPK):
                acc = acc + ws[k] * prow_v[k, pl.ds(j * 16, 16)]
            mrow_v[pl.ds(j * 16, 16)] = acc
        pltpu.sync_copy(mrow_v, mixed_hbm.at[base + r])
        cpz.wait()
        plsc.store_scatter(zbuf, [idx16], zero16, mask=m8)
        return carry

    lax.fori_loop(0, ROWS_PER_W, row_step, 0)


def _sc_scatter_mix(topv, topi, prompts):
    mesh = plsc.VectorSubcoreMesh(core_axis_name="c", subcore_axis_name="s")
    fn = pl.kernel(
        _sc_body,
        out_type=[
            jax.ShapeDtypeStruct((N_FEAT, N_CLASSES), jnp.float32),
            jax.ShapeDtypeStruct((N_FEAT, DIM), jnp.float32),
        ],
        mesh=mesh,
        scratch_types=[
            pltpu.VMEM((ROWS_PER_W * TOPK,), jnp.float32),
            pltpu.VMEM((ROWS_PER_W * TOPK,), jnp.int32),
            pltpu.VMEM((N_CLASSES,), jnp.float32),
            pltpu.VMEM((16,), jnp.int32),
            pltpu.VMEM((16, DIM), jnp.float32),
            pltpu.VMEM((DIM,), jnp.float32),
            pltpu.SemaphoreType.DMA,
            pltpu.SemaphoreType.DMA,
        ],
        compiler_params=pltpu.CompilerParams(needs_layout_passes=False),
    )
    return fn(topv.reshape(-1), topi.reshape(-1), prompts)


def kernel(feat, prompts, topk):
    del topk
    topv, topi = _topk_tc(feat, prompts)
    probs, mixed = _sc_scatter_mix(topv, topi, prompts)
    return (mixed, probs)
